# baseline (device time: 142715 ns/iter reference)
import jax
import jax.numpy as jnp
from jax import lax
from jax.experimental import pallas as pl
from jax.experimental.pallas import tpu as pltpu

N_DEV = 8
N_TOK = 2048
D = 1024
E_LOCAL = 8
N_EXP = 64
CHUNK = N_TOK // N_DEV
N_STEPS = 2 * (N_DEV - 1)


def _compute_body(x_ref, rw_ref, idx_ref, ew_ref, sw_ref, out_ref, gate_ref):
    e = pl.program_id(0)
    d = lax.axis_index("i")

    @pl.when(e == 0)
    def _():
        scores = jnp.dot(
            x_ref[:, :], rw_ref[:, :], preferred_element_type=jnp.float32
        )
        m = jnp.max(scores, axis=-1, keepdims=True)
        p = jnp.exp(scores - m)
        probs = p / jnp.sum(p, axis=-1, keepdims=True)
        sel = (
            lax.broadcasted_iota(jnp.int32, (N_TOK, N_EXP), 1) == idx_ref[:, :]
        )
        gate_ref[:, :] = jnp.sum(
            jnp.where(sel, probs, 0.0), axis=-1, keepdims=True
        )
        out_ref[:, :] = jnp.zeros_like(out_ref)
        row0 = d * CHUNK
        out_ref[pl.ds(row0, CHUNK), :] = jnp.dot(
            x_ref[pl.ds(row0, CHUNK), :],
            sw_ref[:, :],
            preferred_element_type=jnp.float32,
        )

    g = d * E_LOCAL + e
    w = jnp.where(idx_ref[:, :] == g, gate_ref[:, :], 0.0)
    y = jnp.dot(x_ref[:, :], ew_ref[0, :, :], preferred_element_type=jnp.float32)
    out_ref[:, :] += w * y


_PARTS = (
    (0, 96, (4, 2, 1)),
    (768, 96, (2, 1, 4)),
    (1536, 64, (1, 4, 2)),
)
_RS_SLOT_BASE = (0, 4, 6)
_RS_PART_BASE = (0, 672, 1344)
_N_SEMS = 42


def _subset_sums(masks):
    out = [0]
    for m in masks:
        out = out + [s + m for s in out]
    return out


def _allreduce_body(p_ref, out_ref, rs_ref, send_sems, recv_sems):
    d = lax.axis_index("i")
    bx = (d ^ (d >> 1)) & 1
    by = (d >> 1) & 1
    bz = (d >> 2) & 1
    n = bx + 2 * by + 4 * bz

    def pos_of(nn):
        px = nn & 1
        py = (nn >> 1) & 1
        pz = (nn >> 2) & 1
        return 4 * pz + 2 * py + (px ^ py)

    partner_pos = {m: pos_of(n ^ m) for m in (1, 2, 4)}

    out_ref[:, :] = p_ref[:, :]

    barrier = pltpu.get_barrier_semaphore()
    for m in (1, 2, 4):
        pl.semaphore_signal(
            barrier,
            inc=1,
            device_id=(partner_pos[m],),
            device_id_type=pl.DeviceIdType.MESH,
        )
    pl.semaphore_wait(barrier, 3)

    sem_ctr = [0]

    def next_sem():
        i = sem_ctr[0]
        sem_ctr[0] += 1
        return i

    for j in range(3):
        stage = []
        for k, (base, sub, masks) in enumerate(_PARTS):
            mj = masks[j]
            cm = sum(masks[:j])
            send_cons = (n & cm) | ((n & mj) ^ mj)
            recv_cons = n & (cm | mj)
            for idx, f in enumerate(_subset_sums(masks[j + 1 :])):
                si = next_sem()
                slot = _RS_SLOT_BASE[j] + idx
                rdma = pltpu.make_async_remote_copy(
                    src_ref=out_ref.at[pl.ds((base // sub + send_cons + f) * sub, sub), :],
                    dst_ref=rs_ref.at[k, pl.ds(slot * sub, sub), :],
                    send_sem=send_sems.at[si],
                    recv_sem=recv_sems.at[si],
                    device_id=(partner_pos[mj],),
                    device_id_type=pl.DeviceIdType.MESH,
                )
                rdma.start()
                stage.append((rdma, k, base, sub, recv_cons + f, slot))
        for rdma, k, base, sub, v, slot in stage:
            rdma.wait()
            out_ref[pl.ds(base + v * sub, sub), :] += rs_ref[
                k, pl.ds(slot * sub, sub), :
            ]

    for r in range(3):
        stage = []
        for base, sub, masks in _PARTS:
            for f in _subset_sums(masks[3 - r :]):
                si = next_sem()
                row = base + (jnp.bitwise_xor(n, f)) * sub
                rdma = pltpu.make_async_remote_copy(
                    src_ref=out_ref.at[pl.ds(row, sub), :],
                    dst_ref=out_ref.at[pl.ds(row, sub), :],
                    send_sem=send_sems.at[si],
                    recv_sem=recv_sems.at[si],
                    device_id=(partner_pos[masks[2 - r]],),
                    device_id_type=pl.DeviceIdType.MESH,
                )
                rdma.start()
                stage.append(rdma)
        for rdma in stage:
            rdma.wait()


def _allreduce_body_ring(p_ref, out_ref, comm_ref, send_sems, recv_sems):
    d = lax.axis_index("i")
    left = lax.rem(d + N_DEV - 1, N_DEV)
    right = lax.rem(d + 1, N_DEV)

    barrier = pltpu.get_barrier_semaphore()
    for nbr in (left, right):
        pl.semaphore_signal(
            barrier, inc=1, device_id=(nbr,), device_id_type=pl.DeviceIdType.MESH
        )
    pl.semaphore_wait(barrier, 2)

    comm_ref[0, :, :] = p_ref[pl.ds(d * CHUNK, CHUNK), :]
    for k in range(N_STEPS):
        if 1 <= k <= N_DEV - 1:
            c = lax.rem(d - k + 2 * N_DEV, N_DEV)
            comm_ref[k, :, :] += p_ref[pl.ds(c * CHUNK, CHUNK), :]
        if k == N_DEV - 1:
            c = lax.rem(d + 1, N_DEV)
            out_ref[pl.ds(c * CHUNK, CHUNK), :] = comm_ref[k, :, :]
        rdma = pltpu.make_async_remote_copy(
            src_ref=comm_ref.at[k],
            dst_ref=comm_ref.at[k + 1],
            send_sem=send_sems.at[k],
            recv_sem=recv_sems.at[k],
            device_id=(right,),
            device_id_type=pl.DeviceIdType.MESH,
        )
        rdma.start()
        rdma.wait()
        if k >= N_DEV - 1:
            c = lax.rem(d - (k - (N_DEV - 1)) + 2 * N_DEV, N_DEV)
            out_ref[pl.ds(c * CHUNK, CHUNK), :] = comm_ref[k + 1, :, :]


def _fused_body(
    x_ref,
    rw_ref,
    idx_ref,
    ew_ref,
    sw_ref,
    out_ref,
    rs_ref,
    gate_ref,
    send_sems,
    recv_sems,
):
    d = lax.axis_index("i")
    bx = (d ^ (d >> 1)) & 1
    by = (d >> 1) & 1
    bz = (d >> 2) & 1
    n = bx + 2 * by + 4 * bz

    def pos_of(nn):
        px = nn & 1
        py = (nn >> 1) & 1
        pz = (nn >> 2) & 1
        return 4 * pz + 2 * py + (px ^ py)

    partner_pos = {m: pos_of(n ^ m) for m in (1, 2, 4)}

    scores = jnp.dot(x_ref[:, :], rw_ref[:, :], preferred_element_type=jnp.float32)
    mx = jnp.max(scores, axis=-1, keepdims=True)
    p = jnp.exp(scores - mx)
    probs = p / jnp.sum(p, axis=-1, keepdims=True)
    sel = lax.broadcasted_iota(jnp.int32, (N_TOK, N_EXP), 1) == idx_ref[:, :]
    gate_ref[:, :] = jnp.sum(jnp.where(sel, probs, 0.0), axis=-1, keepdims=True)

    barrier = pltpu.get_barrier_semaphore()
    for m in (1, 2, 4):
        pl.semaphore_signal(
            barrier,
            inc=1,
            device_id=(partner_pos[m],),
            device_id_type=pl.DeviceIdType.MESH,
        )
    pl.semaphore_wait(barrier, 3)

    sem_ctr = [0]

    def next_sem():
        i = sem_ctr[0]
        sem_ctr[0] += 1
        return i

    def rs_issue(k, j):
        base, sub, masks = _PARTS[k]
        mj = masks[j]
        cm = sum(masks[:j])
        send_cons = (n & cm) | ((n & mj) ^ mj)
        recv_cons = n & (cm | mj)
        lst = []
        for idx, f in enumerate(_subset_sums(masks[j + 1 :])):
            si = next_sem()
            slot = _RS_SLOT_BASE[j] + idx
            rdma = pltpu.make_async_remote_copy(
                src_ref=out_ref.at[pl.ds(base + (send_cons + f) * sub, sub), :],
                dst_ref=rs_ref.at[pl.ds(_RS_PART_BASE[k] + slot * sub, sub), :],
                send_sem=send_sems.at[si],
                recv_sem=recv_sems.at[si],
                device_id=(partner_pos[mj],),
                device_id_type=pl.DeviceIdType.MESH,
            )
            rdma.start()
            lst.append((rdma, recv_cons + f, slot))
        return lst

    def rs_wait_add(k, lst):
        base, sub, _ = _PARTS[k]
        for rdma, v, slot in lst:
            rdma.wait()
            out_ref[pl.ds(base + v * sub, sub), :] += rs_ref[
                pl.ds(_RS_PART_BASE[k] + slot * sub, sub), :
            ]

    def ag_issue(k, r):
        base, sub, masks = _PARTS[k]
        lst = []
        for f in _subset_sums(masks[3 - r :]):
            si = next_sem()
            row = base + (n ^ f) * sub
            rdma = pltpu.make_async_remote_copy(
                src_ref=out_ref.at[pl.ds(row, sub), :],
                dst_ref=out_ref.at[pl.ds(row, sub), :],
                send_sem=send_sems.at[si],
                recv_sem=recv_sems.at[si],
                device_id=(partner_pos[masks[2 - r]],),
                device_id_type=pl.DeviceIdType.MESH,
            )
            rdma.start()
            lst.append(rdma)
        return lst

    rs_pend = {}
    for k, (base, sub, _) in enumerate(_PARTS):
        rows = 8 * sub
        for e in range(E_LOCAL):
            g = d * E_LOCAL + e
            w = jnp.where(
                idx_ref[pl.ds(base, rows), :] == g,
                gate_ref[pl.ds(base, rows), :],
                0.0,
            )
            y = jnp.dot(
                x_ref[pl.ds(base, rows), :],
                ew_ref[e, :, :],
                preferred_element_type=jnp.float32,
            )
            if e == 0:
                out_ref[pl.ds(base, rows), :] = w * y
            else:
                out_ref[pl.ds(base, rows), :] += w * y
        row0 = d * CHUNK

        @pl.when((row0 >= base) & (row0 < base + rows))
        def _():
            out_ref[pl.ds(row0, CHUNK), :] += jnp.dot(
                x_ref[pl.ds(row0, CHUNK), :],
                sw_ref[:, :],
                preferred_element_type=jnp.float32,
            )

        rs_pend[k] = rs_issue(k, 0)

    for j in (1, 2):
        for k in range(3):
            rs_wait_add(k, rs_pend[k])
            rs_pend[k] = rs_issue(k, j)
    ag_pend = {}
    for k in range(3):
        rs_wait_add(k, rs_pend[k])
        ag_pend[k] = ag_issue(k, 0)
    for r in (1, 2):
        for k in range(3):
            for rdma in ag_pend[k]:
                rdma.wait()
            ag_pend[k] = ag_issue(k, r)
    for k in range(3):
        for rdma in ag_pend[k]:
            rdma.wait()


def _fused(x, router_W, route_idx, expert_W, shared_W):
    return pl.pallas_call(
        _fused_body,
        out_shape=jax.ShapeDtypeStruct((N_TOK, D), jnp.float32),
        in_specs=[pl.BlockSpec(memory_space=pltpu.VMEM)] * 5,
        out_specs=pl.BlockSpec(memory_space=pltpu.VMEM),
        scratch_shapes=[
            pltpu.VMEM((1792, D), jnp.float32),
            pltpu.VMEM((N_TOK, 1), jnp.float32),
            pltpu.SemaphoreType.DMA((_N_SEMS,)),
            pltpu.SemaphoreType.DMA((_N_SEMS,)),
        ],
        compiler_params=pltpu.CompilerParams(
            collective_id=0, vmem_limit_bytes=120 * 1024 * 1024
        ),
    )(x, router_W, route_idx, expert_W, shared_W)


import os
_AR_ONLY = os.environ.get("SCBAND_AR_ONLY") == "1"
_UNFUSED = os.environ.get("SCBAND_UNFUSED") == "1"


def kernel(x, router_W, route_idx, expert_W, shared_W):
    if _AR_ONLY:
        return _allreduce(x)
    if not _UNFUSED:
        return _fused(x, router_W, route_idx, expert_W, shared_W)
    partial = pl.pallas_call(
        _compute_body,
        grid=(E_LOCAL,),
        in_specs=[
            pl.BlockSpec((N_TOK, D), lambda e: (0, 0)),
            pl.BlockSpec((D, N_EXP), lambda e: (0, 0)),
            pl.BlockSpec((N_TOK, 1), lambda e: (0, 0)),
            pl.BlockSpec((1, D, D), lambda e: (e, 0, 0)),
            pl.BlockSpec((D, D), lambda e: (0, 0)),
        ],
        out_specs=pl.BlockSpec((N_TOK, D), lambda e: (0, 0)),
        out_shape=jax.ShapeDtypeStruct((N_TOK, D), jnp.float32),
        scratch_shapes=[pltpu.VMEM((N_TOK, 1), jnp.float32)],
        compiler_params=pltpu.CompilerParams(
            dimension_semantics=("arbitrary",),
        ),
    )(x, router_W, route_idx, expert_W, shared_W)

    return _allreduce(partial)


def _allreduce(partial):
    return pl.pallas_call(
        _allreduce_body,
        out_shape=jax.ShapeDtypeStruct((N_TOK, D), jnp.float32),
        in_specs=[pl.BlockSpec(memory_space=pltpu.VMEM)],
        out_specs=pl.BlockSpec(memory_space=pltpu.VMEM),
        scratch_shapes=[
            pltpu.VMEM((3, 7 * 96, D), jnp.float32),
            pltpu.SemaphoreType.DMA((_N_SEMS,)),
            pltpu.SemaphoreType.DMA((_N_SEMS,)),
        ],
        compiler_params=pltpu.CompilerParams(collective_id=0),
    )(partial)


# device time: 102853 ns/iter; 1.3876x vs baseline; 1.3876x over previous
import jax
import jax.numpy as jnp
from jax import lax
from jax.experimental import pallas as pl
from jax.experimental.pallas import tpu as pltpu

N_DEV = 8
N_TOK = 2048
D = 1024
E_LOCAL = 8
N_EXP = 64
CHUNK = N_TOK // N_DEV
N_STEPS = 2 * (N_DEV - 1)


def _compute_body(x_ref, rw_ref, idx_ref, ew_ref, sw_ref, out_ref, gate_ref, xb_ref):
    e = pl.program_id(0)
    d = lax.axis_index("i")

    @pl.when(e == 0)
    def _():
        scores = jnp.dot(
            x_ref[:, :], rw_ref[:, :], preferred_element_type=jnp.float32
        )
        m = jnp.max(scores, axis=-1, keepdims=True)
        p = jnp.exp(scores - m)
        probs = p / jnp.sum(p, axis=-1, keepdims=True)
        sel = (
            lax.broadcasted_iota(jnp.int32, (N_TOK, N_EXP), 1) == idx_ref[:, :]
        )
        gate_ref[:, :] = jnp.sum(
            jnp.where(sel, probs, 0.0), axis=-1, keepdims=True
        )
        xb_ref[:, :] = x_ref[:, :].astype(jnp.bfloat16)
        out_ref[:, :] = jnp.zeros_like(out_ref)
        row0 = d * CHUNK
        out_ref[pl.ds(row0, CHUNK), :] = jnp.dot(
            xb_ref[pl.ds(row0, CHUNK), :],
            sw_ref[:, :].astype(jnp.bfloat16),
            preferred_element_type=jnp.float32,
        ).astype(jnp.bfloat16)

    g = d * E_LOCAL + e
    w = jnp.where(idx_ref[:, :] == g, gate_ref[:, :], 0.0)
    y = jnp.dot(
        xb_ref[:, :],
        ew_ref[0, :, :].astype(jnp.bfloat16),
        preferred_element_type=jnp.float32,
    )
    out_ref[:, :] = (out_ref[:, :].astype(jnp.float32) + w * y).astype(
        jnp.bfloat16
    )


_PARTS = (
    (0, 96, (4, 2, 1)),
    (768, 96, (2, 1, 4)),
    (1536, 64, (1, 4, 2)),
)
_RS_SLOT_BASE = (0, 4, 6)
_RS_PART_BASE = (0, 672, 1344)
_N_SEMS = 42


def _subset_sums(masks):
    out = [0]
    for m in masks:
        out = out + [s + m for s in out]
    return out


def _allreduce_body(p_ref, out_ref, acc_ref, rs_ref, send_sems, recv_sems):
    d = lax.axis_index("i")
    bx = (d ^ (d >> 1)) & 1
    by = (d >> 1) & 1
    bz = (d >> 2) & 1
    n = bx + 2 * by + 4 * bz

    def pos_of(nn):
        px = nn & 1
        py = (nn >> 1) & 1
        pz = (nn >> 2) & 1
        return 4 * pz + 2 * py + (px ^ py)

    partner_pos = {m: pos_of(n ^ m) for m in (1, 2, 4)}

    acc_ref[:, :] = p_ref[:, :]

    barrier = pltpu.get_barrier_semaphore()
    for m in (1, 2, 4):
        pl.semaphore_signal(
            barrier,
            inc=1,
            device_id=(partner_pos[m],),
            device_id_type=pl.DeviceIdType.MESH,
        )
    pl.semaphore_wait(barrier, 3)

    sem_ctr = [0]

    def next_sem():
        i = sem_ctr[0]
        sem_ctr[0] += 1
        return i

    for j in range(3):
        stage = []
        for k, (base, sub, masks) in enumerate(_PARTS):
            mj = masks[j]
            cm = sum(masks[:j])
            send_cons = (n & cm) | ((n & mj) ^ mj)
            recv_cons = n & (cm | mj)
            for idx, f in enumerate(_subset_sums(masks[j + 1 :])):
                si = next_sem()
                slot_row = _RS_PART_BASE[k] + (_RS_SLOT_BASE[j] + idx) * sub
                rdma = pltpu.make_async_remote_copy(
                    src_ref=acc_ref.at[pl.ds(base + (send_cons + f) * sub, sub), :],
                    dst_ref=rs_ref.at[pl.ds(slot_row, sub), :],
                    send_sem=send_sems.at[si],
                    recv_sem=recv_sems.at[si],
                    device_id=(partner_pos[mj],),
                    device_id_type=pl.DeviceIdType.MESH,
                )
                rdma.start()
                stage.append((rdma, base, sub, recv_cons + f, slot_row))
        for rdma, base, sub, v, slot_row in stage:
            rdma.wait()
            acc_ref[pl.ds(base + v * sub, sub), :] += rs_ref[
                pl.ds(slot_row, sub), :
            ]

    for r in range(3):
        stage = []
        for base, sub, masks in _PARTS:
            for f in _subset_sums(masks[3 - r :]):
                si = next_sem()
                row = base + (n ^ f) * sub
                rdma = pltpu.make_async_remote_copy(
                    src_ref=acc_ref.at[pl.ds(row, sub), :],
                    dst_ref=acc_ref.at[pl.ds(row, sub), :],
                    send_sem=send_sems.at[si],
                    recv_sem=recv_sems.at[si],
                    device_id=(partner_pos[masks[2 - r]],),
                    device_id_type=pl.DeviceIdType.MESH,
                )
                rdma.start()
                stage.append(rdma)
        for rdma in stage:
            rdma.wait()

    out_ref[:, :] = acc_ref[:, :].astype(jnp.float32)


def _allreduce_body_ring(p_ref, out_ref, comm_ref, send_sems, recv_sems):
    d = lax.axis_index("i")
    left = lax.rem(d + N_DEV - 1, N_DEV)
    right = lax.rem(d + 1, N_DEV)

    barrier = pltpu.get_barrier_semaphore()
    for nbr in (left, right):
        pl.semaphore_signal(
            barrier, inc=1, device_id=(nbr,), device_id_type=pl.DeviceIdType.MESH
        )
    pl.semaphore_wait(barrier, 2)

    comm_ref[0, :, :] = p_ref[pl.ds(d * CHUNK, CHUNK), :]
    for k in range(N_STEPS):
        if 1 <= k <= N_DEV - 1:
            c = lax.rem(d - k + 2 * N_DEV, N_DEV)
            comm_ref[k, :, :] += p_ref[pl.ds(c * CHUNK, CHUNK), :]
        if k == N_DEV - 1:
            c = lax.rem(d + 1, N_DEV)
            out_ref[pl.ds(c * CHUNK, CHUNK), :] = comm_ref[k, :, :]
        rdma = pltpu.make_async_remote_copy(
            src_ref=comm_ref.at[k],
            dst_ref=comm_ref.at[k + 1],
            send_sem=send_sems.at[k],
            recv_sem=recv_sems.at[k],
            device_id=(right,),
            device_id_type=pl.DeviceIdType.MESH,
        )
        rdma.start()
        rdma.wait()
        if k >= N_DEV - 1:
            c = lax.rem(d - (k - (N_DEV - 1)) + 2 * N_DEV, N_DEV)
            out_ref[pl.ds(c * CHUNK, CHUNK), :] = comm_ref[k + 1, :, :]


def _fused_body(
    x_ref,
    rw_ref,
    idx_ref,
    ew_ref,
    sw_ref,
    out_ref,
    rs_ref,
    gate_ref,
    send_sems,
    recv_sems,
):
    d = lax.axis_index("i")
    bx = (d ^ (d >> 1)) & 1
    by = (d >> 1) & 1
    bz = (d >> 2) & 1
    n = bx + 2 * by + 4 * bz

    def pos_of(nn):
        px = nn & 1
        py = (nn >> 1) & 1
        pz = (nn >> 2) & 1
        return 4 * pz + 2 * py + (px ^ py)

    partner_pos = {m: pos_of(n ^ m) for m in (1, 2, 4)}

    scores = jnp.dot(x_ref[:, :], rw_ref[:, :], preferred_element_type=jnp.float32)
    mx = jnp.max(scores, axis=-1, keepdims=True)
    p = jnp.exp(scores - mx)
    probs = p / jnp.sum(p, axis=-1, keepdims=True)
    sel = lax.broadcasted_iota(jnp.int32, (N_TOK, N_EXP), 1) == idx_ref[:, :]
    gate_ref[:, :] = jnp.sum(jnp.where(sel, probs, 0.0), axis=-1, keepdims=True)

    barrier = pltpu.get_barrier_semaphore()
    for m in (1, 2, 4):
        pl.semaphore_signal(
            barrier,
            inc=1,
            device_id=(partner_pos[m],),
            device_id_type=pl.DeviceIdType.MESH,
        )
    pl.semaphore_wait(barrier, 3)

    sem_ctr = [0]

    def next_sem():
        i = sem_ctr[0]
        sem_ctr[0] += 1
        return i

    def rs_issue(k, j):
        base, sub, masks = _PARTS[k]
        mj = masks[j]
        cm = sum(masks[:j])
        send_cons = (n & cm) | ((n & mj) ^ mj)
        recv_cons = n & (cm | mj)
        lst = []
        for idx, f in enumerate(_subset_sums(masks[j + 1 :])):
            si = next_sem()
            slot = _RS_SLOT_BASE[j] + idx
            rdma = pltpu.make_async_remote_copy(
                src_ref=out_ref.at[pl.ds(base + (send_cons + f) * sub, sub), :],
                dst_ref=rs_ref.at[pl.ds(_RS_PART_BASE[k] + slot * sub, sub), :],
                send_sem=send_sems.at[si],
                recv_sem=recv_sems.at[si],
                device_id=(partner_pos[mj],),
                device_id_type=pl.DeviceIdType.MESH,
            )
            rdma.start()
            lst.append((rdma, recv_cons + f, slot))
        return lst

    def rs_wait_add(k, lst):
        base, sub, _ = _PARTS[k]
        for rdma, v, slot in lst:
            rdma.wait()
            out_ref[pl.ds(base + v * sub, sub), :] += rs_ref[
                pl.ds(_RS_PART_BASE[k] + slot * sub, sub), :
            ]

    def ag_issue(k, r):
        base, sub, masks = _PARTS[k]
        lst = []
        for f in _subset_sums(masks[3 - r :]):
            si = next_sem()
            row = base + (n ^ f) * sub
            rdma = pltpu.make_async_remote_copy(
                src_ref=out_ref.at[pl.ds(row, sub), :],
                dst_ref=out_ref.at[pl.ds(row, sub), :],
                send_sem=send_sems.at[si],
                recv_sem=recv_sems.at[si],
                device_id=(partner_pos[masks[2 - r]],),
                device_id_type=pl.DeviceIdType.MESH,
            )
            rdma.start()
            lst.append(rdma)
        return lst

    rs_pend = {}
    for k, (base, sub, _) in enumerate(_PARTS):
        rows = 8 * sub
        for e in range(E_LOCAL):
            g = d * E_LOCAL + e
            w = jnp.where(
                idx_ref[pl.ds(base, rows), :] == g,
                gate_ref[pl.ds(base, rows), :],
                0.0,
            )
            y = jnp.dot(
                x_ref[pl.ds(base, rows), :],
                ew_ref[e, :, :],
                preferred_element_type=jnp.float32,
            )
            if e == 0:
                out_ref[pl.ds(base, rows), :] = w * y
            else:
                out_ref[pl.ds(base, rows), :] += w * y
        row0 = d * CHUNK

        @pl.when((row0 >= base) & (row0 < base + rows))
        def _():
            out_ref[pl.ds(row0, CHUNK), :] += jnp.dot(
                x_ref[pl.ds(row0, CHUNK), :],
                sw_ref[:, :],
                preferred_element_type=jnp.float32,
            )

        rs_pend[k] = rs_issue(k, 0)

    for j in (1, 2):
        for k in range(3):
            rs_wait_add(k, rs_pend[k])
            rs_pend[k] = rs_issue(k, j)
    ag_pend = {}
    for k in range(3):
        rs_wait_add(k, rs_pend[k])
        ag_pend[k] = ag_issue(k, 0)
    for r in (1, 2):
        for k in range(3):
            for rdma in ag_pend[k]:
                rdma.wait()
            ag_pend[k] = ag_issue(k, r)
    for k in range(3):
        for rdma in ag_pend[k]:
            rdma.wait()


def _fused(x, router_W, route_idx, expert_W, shared_W):
    return pl.pallas_call(
        _fused_body,
        out_shape=jax.ShapeDtypeStruct((N_TOK, D), jnp.float32),
        in_specs=[pl.BlockSpec(memory_space=pltpu.VMEM)] * 5,
        out_specs=pl.BlockSpec(memory_space=pltpu.VMEM),
        scratch_shapes=[
            pltpu.VMEM((1792, D), jnp.float32),
            pltpu.VMEM((N_TOK, 1), jnp.float32),
            pltpu.SemaphoreType.DMA((_N_SEMS,)),
            pltpu.SemaphoreType.DMA((_N_SEMS,)),
        ],
        compiler_params=pltpu.CompilerParams(
            collective_id=0, vmem_limit_bytes=120 * 1024 * 1024
        ),
    )(x, router_W, route_idx, expert_W, shared_W)


import os
_AR_ONLY = os.environ.get("SCBAND_AR_ONLY") == "1"
_FUSED = os.environ.get("SCBAND_FUSED") == "1"


def kernel(x, router_W, route_idx, expert_W, shared_W):
    if _AR_ONLY:
        return _allreduce(x.astype(jnp.bfloat16))
    if _FUSED:
        return _fused(x, router_W, route_idx, expert_W, shared_W)
    partial = pl.pallas_call(
        _compute_body,
        grid=(E_LOCAL,),
        in_specs=[
            pl.BlockSpec((N_TOK, D), lambda e: (0, 0)),
            pl.BlockSpec((D, N_EXP), lambda e: (0, 0)),
            pl.BlockSpec((N_TOK, 1), lambda e: (0, 0)),
            pl.BlockSpec((1, D, D), lambda e: (e, 0, 0)),
            pl.BlockSpec((D, D), lambda e: (0, 0)),
        ],
        out_specs=pl.BlockSpec((N_TOK, D), lambda e: (0, 0)),
        out_shape=jax.ShapeDtypeStruct((N_TOK, D), jnp.bfloat16),
        scratch_shapes=[
            pltpu.VMEM((N_TOK, 1), jnp.float32),
            pltpu.VMEM((N_TOK, D), jnp.bfloat16),
        ],
        compiler_params=pltpu.CompilerParams(
            dimension_semantics=("arbitrary",),
        ),
    )(x, router_W, route_idx, expert_W, shared_W)

    return _allreduce(partial)


def _allreduce(partial):
    return pl.pallas_call(
        _allreduce_body,
        out_shape=jax.ShapeDtypeStruct((N_TOK, D), jnp.float32),
        in_specs=[pl.BlockSpec(memory_space=pltpu.VMEM)],
        out_specs=pl.BlockSpec(memory_space=pltpu.VMEM),
        scratch_shapes=[
            pltpu.VMEM((N_TOK, D), jnp.bfloat16),
            pltpu.VMEM((1792, D), jnp.bfloat16),
            pltpu.SemaphoreType.DMA((_N_SEMS,)),
            pltpu.SemaphoreType.DMA((_N_SEMS,)),
        ],
        compiler_params=pltpu.CompilerParams(collective_id=0),
    )(partial)


# device time: 99561 ns/iter; 1.4334x vs baseline; 1.0331x over previous
import jax
import jax.numpy as jnp
from jax import lax
from jax.experimental import pallas as pl
from jax.experimental.pallas import tpu as pltpu

N_DEV = 8
N_TOK = 2048
D = 1024
E_LOCAL = 8
N_EXP = 64
CHUNK = N_TOK // N_DEV
N_STEPS = 2 * (N_DEV - 1)


def _compute_body(
    x_ref, rw_ref, idx_ref, ew_ref, sw_ref, out_ref, gate_ref, xb_ref, acc_ref
):
    e = pl.program_id(0)
    d = lax.axis_index("i")

    @pl.when(e == 0)
    def _():
        scores = jnp.dot(
            x_ref[:, :], rw_ref[:, :], preferred_element_type=jnp.float32
        )
        m = jnp.max(scores, axis=-1, keepdims=True)
        p = jnp.exp(scores - m)
        probs = p / jnp.sum(p, axis=-1, keepdims=True)
        sel = (
            lax.broadcasted_iota(jnp.int32, (N_TOK, N_EXP), 1) == idx_ref[:, :]
        )
        gate_ref[:, :] = jnp.sum(
            jnp.where(sel, probs, 0.0), axis=-1, keepdims=True
        )
        xb_ref[:, :] = x_ref[:, :].astype(jnp.bfloat16)
        acc_ref[:, :] = jnp.zeros_like(acc_ref)
        row0 = d * CHUNK
        acc_ref[pl.ds(row0, CHUNK), :] = jnp.dot(
            xb_ref[pl.ds(row0, CHUNK), :],
            sw_ref[:, :].astype(jnp.bfloat16),
            preferred_element_type=jnp.float32,
        )

    g = d * E_LOCAL + e
    w = jnp.where(idx_ref[:, :] == g, gate_ref[:, :], 0.0)
    y = jnp.dot(
        xb_ref[:, :],
        ew_ref[0, :, :].astype(jnp.bfloat16),
        preferred_element_type=jnp.float32,
    )
    acc_ref[:, :] += w * y

    @pl.when(e == E_LOCAL - 1)
    def _():
        out_ref[:, :] = acc_ref[:, :].astype(jnp.bfloat16)


_PARTS = (
    (0, 96, (4, 2, 1)),
    (768, 96, (2, 1, 4)),
    (1536, 64, (1, 4, 2)),
)
_RS_SLOT_BASE = (0, 4, 6)
_RS_PART_BASE = (0, 672, 1344)
_N_SEMS = 42


def _subset_sums(masks):
    out = [0]
    for m in masks:
        out = out + [s + m for s in out]
    return out


def _allreduce_body(p_ref, out_ref, rs_ref, send_sems, recv_sems):
    d = lax.axis_index("i")
    bx = (d ^ (d >> 1)) & 1
    by = (d >> 1) & 1
    bz = (d >> 2) & 1
    n = bx + 2 * by + 4 * bz

    def pos_of(nn):
        px = nn & 1
        py = (nn >> 1) & 1
        pz = (nn >> 2) & 1
        return 4 * pz + 2 * py + (px ^ py)

    partner_pos = {m: pos_of(n ^ m) for m in (1, 2, 4)}

    barrier = pltpu.get_barrier_semaphore()
    for m in (1, 2, 4):
        pl.semaphore_signal(
            barrier,
            inc=1,
            device_id=(partner_pos[m],),
            device_id_type=pl.DeviceIdType.MESH,
        )
    pl.semaphore_wait(barrier, 3)

    sem_ctr = [0]

    def next_sem():
        i = sem_ctr[0]
        sem_ctr[0] += 1
        return i

    def rs_issue(k, j):
        base, sub, masks = _PARTS[k]
        mj = masks[j]
        cm = sum(masks[:j])
        send_cons = (n & cm) | ((n & mj) ^ mj)
        recv_cons = n & (cm | mj)
        src = p_ref if j == 0 else out_ref
        lst = []
        for idx, f in enumerate(_subset_sums(masks[j + 1 :])):
            si = next_sem()
            slot_row = _RS_PART_BASE[k] + (_RS_SLOT_BASE[j] + idx) * sub
            rdma = pltpu.make_async_remote_copy(
                src_ref=src.at[pl.ds(base + (send_cons + f) * sub, sub), :],
                dst_ref=rs_ref.at[pl.ds(slot_row, sub), :],
                send_sem=send_sems.at[si],
                recv_sem=recv_sems.at[si],
                device_id=(partner_pos[mj],),
                device_id_type=pl.DeviceIdType.MESH,
            )
            rdma.start()
            lst.append((rdma, recv_cons + f, slot_row))
        return lst

    def rs_wait_add(k, j, lst):
        base, sub, _ = _PARTS[k]
        for rdma, v, slot_row in lst:
            rdma.wait()
            rows = pl.ds(base + v * sub, sub)
            recv = rs_ref[pl.ds(slot_row, sub), :]
            if j == 0:
                out_ref[rows, :] = p_ref[rows, :] + recv
            else:
                out_ref[rows, :] += recv

    def ag_issue(k, r):
        base, sub, masks = _PARTS[k]
        lst = []
        for f in _subset_sums(masks[3 - r :]):
            si = next_sem()
            row = base + (n ^ f) * sub
            rdma = pltpu.make_async_remote_copy(
                src_ref=out_ref.at[pl.ds(row, sub), :],
                dst_ref=out_ref.at[pl.ds(row, sub), :],
                send_sem=send_sems.at[si],
                recv_sem=recv_sems.at[si],
                device_id=(partner_pos[masks[2 - r]],),
                device_id_type=pl.DeviceIdType.MESH,
            )
            rdma.start()
            lst.append(rdma)
        return lst

    pend = {k: rs_issue(k, 0) for k in range(3)}
    for j in (1, 2):
        for k in range(3):
            rs_wait_add(k, j - 1, pend[k])
            pend[k] = rs_issue(k, j)
    ag = {}
    for k in range(3):
        rs_wait_add(k, 2, pend[k])
        ag[k] = ag_issue(k, 0)
    for r in (1, 2):
        for k in range(3):
            for rdma in ag[k]:
                rdma.wait()
            ag[k] = ag_issue(k, r)
    for k in range(3):
        for rdma in ag[k]:
            rdma.wait()


def _allreduce_body_ring(p_ref, out_ref, comm_ref, send_sems, recv_sems):
    d = lax.axis_index("i")
    left = lax.rem(d + N_DEV - 1, N_DEV)
    right = lax.rem(d + 1, N_DEV)

    barrier = pltpu.get_barrier_semaphore()
    for nbr in (left, right):
        pl.semaphore_signal(
            barrier, inc=1, device_id=(nbr,), device_id_type=pl.DeviceIdType.MESH
        )
    pl.semaphore_wait(barrier, 2)

    comm_ref[0, :, :] = p_ref[pl.ds(d * CHUNK, CHUNK), :]
    for k in range(N_STEPS):
        if 1 <= k <= N_DEV - 1:
            c = lax.rem(d - k + 2 * N_DEV, N_DEV)
            comm_ref[k, :, :] += p_ref[pl.ds(c * CHUNK, CHUNK), :]
        if k == N_DEV - 1:
            c = lax.rem(d + 1, N_DEV)
            out_ref[pl.ds(c * CHUNK, CHUNK), :] = comm_ref[k, :, :]
        rdma = pltpu.make_async_remote_copy(
            src_ref=comm_ref.at[k],
            dst_ref=comm_ref.at[k + 1],
            send_sem=send_sems.at[k],
            recv_sem=recv_sems.at[k],
            device_id=(right,),
            device_id_type=pl.DeviceIdType.MESH,
        )
        rdma.start()
        rdma.wait()
        if k >= N_DEV - 1:
            c = lax.rem(d - (k - (N_DEV - 1)) + 2 * N_DEV, N_DEV)
            out_ref[pl.ds(c * CHUNK, CHUNK), :] = comm_ref[k + 1, :, :]


def _fused_body(
    x_ref,
    rw_ref,
    idx_ref,
    ew_ref,
    sw_ref,
    out_ref,
    rs_ref,
    gate_ref,
    send_sems,
    recv_sems,
):
    d = lax.axis_index("i")
    bx = (d ^ (d >> 1)) & 1
    by = (d >> 1) & 1
    bz = (d >> 2) & 1
    n = bx + 2 * by + 4 * bz

    def pos_of(nn):
        px = nn & 1
        py = (nn >> 1) & 1
        pz = (nn >> 2) & 1
        return 4 * pz + 2 * py + (px ^ py)

    partner_pos = {m: pos_of(n ^ m) for m in (1, 2, 4)}

    scores = jnp.dot(x_ref[:, :], rw_ref[:, :], preferred_element_type=jnp.float32)
    mx = jnp.max(scores, axis=-1, keepdims=True)
    p = jnp.exp(scores - mx)
    probs = p / jnp.sum(p, axis=-1, keepdims=True)
    sel = lax.broadcasted_iota(jnp.int32, (N_TOK, N_EXP), 1) == idx_ref[:, :]
    gate_ref[:, :] = jnp.sum(jnp.where(sel, probs, 0.0), axis=-1, keepdims=True)

    barrier = pltpu.get_barrier_semaphore()
    for m in (1, 2, 4):
        pl.semaphore_signal(
            barrier,
            inc=1,
            device_id=(partner_pos[m],),
            device_id_type=pl.DeviceIdType.MESH,
        )
    pl.semaphore_wait(barrier, 3)

    sem_ctr = [0]

    def next_sem():
        i = sem_ctr[0]
        sem_ctr[0] += 1
        return i

    def rs_issue(k, j):
        base, sub, masks = _PARTS[k]
        mj = masks[j]
        cm = sum(masks[:j])
        send_cons = (n & cm) | ((n & mj) ^ mj)
        recv_cons = n & (cm | mj)
        lst = []
        for idx, f in enumerate(_subset_sums(masks[j + 1 :])):
            si = next_sem()
            slot = _RS_SLOT_BASE[j] + idx
            rdma = pltpu.make_async_remote_copy(
                src_ref=out_ref.at[pl.ds(base + (send_cons + f) * sub, sub), :],
                dst_ref=rs_ref.at[pl.ds(_RS_PART_BASE[k] + slot * sub, sub), :],
                send_sem=send_sems.at[si],
                recv_sem=recv_sems.at[si],
                device_id=(partner_pos[mj],),
                device_id_type=pl.DeviceIdType.MESH,
            )
            rdma.start()
            lst.append((rdma, recv_cons + f, slot))
        return lst

    def rs_wait_add(k, lst):
        base, sub, _ = _PARTS[k]
        for rdma, v, slot in lst:
            rdma.wait()
            out_ref[pl.ds(base + v * sub, sub), :] += rs_ref[
                pl.ds(_RS_PART_BASE[k] + slot * sub, sub), :
            ]

    def ag_issue(k, r):
        base, sub, masks = _PARTS[k]
        lst = []
        for f in _subset_sums(masks[3 - r :]):
            si = next_sem()
            row = base + (n ^ f) * sub
            rdma = pltpu.make_async_remote_copy(
                src_ref=out_ref.at[pl.ds(row, sub), :],
                dst_ref=out_ref.at[pl.ds(row, sub), :],
                send_sem=send_sems.at[si],
                recv_sem=recv_sems.at[si],
                device_id=(partner_pos[masks[2 - r]],),
                device_id_type=pl.DeviceIdType.MESH,
            )
            rdma.start()
            lst.append(rdma)
        return lst

    rs_pend = {}
    for k, (base, sub, _) in enumerate(_PARTS):
        rows = 8 * sub
        for e in range(E_LOCAL):
            g = d * E_LOCAL + e
            w = jnp.where(
                idx_ref[pl.ds(base, rows), :] == g,
                gate_ref[pl.ds(base, rows), :],
                0.0,
            )
            y = jnp.dot(
                x_ref[pl.ds(base, rows), :],
                ew_ref[e, :, :],
                preferred_element_type=jnp.float32,
            )
            if e == 0:
                out_ref[pl.ds(base, rows), :] = w * y
            else:
                out_ref[pl.ds(base, rows), :] += w * y
        row0 = d * CHUNK

        @pl.when((row0 >= base) & (row0 < base + rows))
        def _():
            out_ref[pl.ds(row0, CHUNK), :] += jnp.dot(
                x_ref[pl.ds(row0, CHUNK), :],
                sw_ref[:, :],
                preferred_element_type=jnp.float32,
            )

        rs_pend[k] = rs_issue(k, 0)

    for j in (1, 2):
        for k in range(3):
            rs_wait_add(k, rs_pend[k])
            rs_pend[k] = rs_issue(k, j)
    ag_pend = {}
    for k in range(3):
        rs_wait_add(k, rs_pend[k])
        ag_pend[k] = ag_issue(k, 0)
    for r in (1, 2):
        for k in range(3):
            for rdma in ag_pend[k]:
                rdma.wait()
            ag_pend[k] = ag_issue(k, r)
    for k in range(3):
        for rdma in ag_pend[k]:
            rdma.wait()


def _fused(x, router_W, route_idx, expert_W, shared_W):
    return pl.pallas_call(
        _fused_body,
        out_shape=jax.ShapeDtypeStruct((N_TOK, D), jnp.float32),
        in_specs=[pl.BlockSpec(memory_space=pltpu.VMEM)] * 5,
        out_specs=pl.BlockSpec(memory_space=pltpu.VMEM),
        scratch_shapes=[
            pltpu.VMEM((1792, D), jnp.float32),
            pltpu.VMEM((N_TOK, 1), jnp.float32),
            pltpu.SemaphoreType.DMA((_N_SEMS,)),
            pltpu.SemaphoreType.DMA((_N_SEMS,)),
        ],
        compiler_params=pltpu.CompilerParams(
            collective_id=0, vmem_limit_bytes=120 * 1024 * 1024
        ),
    )(x, router_W, route_idx, expert_W, shared_W)


CAP = 512


def _gcompute_body(
    xg_ref, lid_ref, wg_ref, xs_ref, ew_ref, sw_ref, yg_ref, sh_ref, xgb_ref
):
    e = pl.program_id(0)

    @pl.when(e == 0)
    def _():
        xgb_ref[:, :] = xg_ref[:, :].astype(jnp.bfloat16)
        sh_ref[:, :] = jnp.dot(
            xs_ref[:, :].astype(jnp.bfloat16),
            sw_ref[:, :].astype(jnp.bfloat16),
            preferred_element_type=jnp.float32,
        ).astype(jnp.bfloat16)

    w = jnp.where(lid_ref[:, :] == e, wg_ref[:, :], 0.0)
    y = jnp.dot(
        xgb_ref[:, :],
        ew_ref[0, :, :].astype(jnp.bfloat16),
        preferred_element_type=jnp.float32,
    )

    @pl.when(e == 0)
    def _():
        yg_ref[:, :] = (w * y).astype(jnp.bfloat16)

    @pl.when(e > 0)
    def _():
        yg_ref[:, :] = (yg_ref[:, :].astype(jnp.float32) + w * y).astype(
            jnp.bfloat16
        )


def _gather_compute(x, router_W, route_idx, expert_W, shared_W):
    d = lax.axis_index("i")
    eid = route_idx[:, 0]
    probs = jax.nn.softmax(
        jnp.dot(x, router_W, preferred_element_type=jnp.float32), axis=-1
    )
    gtok = jnp.take_along_axis(probs, route_idx, axis=1)[:, 0]
    mask = (eid // E_LOCAL) == d
    (sel,) = jnp.nonzero(mask, size=CAP, fill_value=0)
    count = jnp.sum(mask)
    sel_safe = jnp.where(jnp.arange(CAP) < count, sel, N_TOK)
    xg = x[sel]
    lid = (eid[sel] - d * E_LOCAL).astype(jnp.int32)
    wg = gtok[sel]
    xs = lax.dynamic_slice(x, (d * CHUNK, 0), (CHUNK, D))

    yg, sh = pl.pallas_call(
        _gcompute_body,
        grid=(E_LOCAL,),
        in_specs=[
            pl.BlockSpec((CAP, D), lambda e: (0, 0)),
            pl.BlockSpec((CAP, 1), lambda e: (0, 0)),
            pl.BlockSpec((CAP, 1), lambda e: (0, 0)),
            pl.BlockSpec((CHUNK, D), lambda e: (0, 0)),
            pl.BlockSpec((1, D, D), lambda e: (e, 0, 0)),
            pl.BlockSpec((D, D), lambda e: (0, 0)),
        ],
        out_specs=(
            pl.BlockSpec((CAP, D), lambda e: (0, 0)),
            pl.BlockSpec((CHUNK, D), lambda e: (0, 0)),
        ),
        out_shape=(
            jax.ShapeDtypeStruct((CAP, D), jnp.bfloat16),
            jax.ShapeDtypeStruct((CHUNK, D), jnp.bfloat16),
        ),
        scratch_shapes=[pltpu.VMEM((CAP, D), jnp.bfloat16)],
        compiler_params=pltpu.CompilerParams(
            dimension_semantics=("arbitrary",),
        ),
    )(xg, lid[:, None], wg[:, None], xs, expert_W, shared_W)

    partial = jnp.zeros((N_TOK, D), jnp.bfloat16).at[sel_safe].set(yg)
    ps = lax.dynamic_slice(partial, (d * CHUNK, 0), (CHUNK, D))
    partial = lax.dynamic_update_slice(partial, ps + sh, (d * CHUNK, 0))
    return _allreduce(partial)


import os
_AR_ONLY = os.environ.get("SCBAND_AR_ONLY") == "1"
_FUSED = os.environ.get("SCBAND_FUSED") == "1"
_GATHER = os.environ.get("SCBAND_GATHER") == "1"


def kernel(x, router_W, route_idx, expert_W, shared_W):
    if _AR_ONLY:
        return _allreduce(x.astype(jnp.bfloat16))
    if _FUSED:
        return _fused(x, router_W, route_idx, expert_W, shared_W)
    if _GATHER:
        return _gather_compute(x, router_W, route_idx, expert_W, shared_W)
    partial = pl.pallas_call(
        _compute_body,
        grid=(E_LOCAL,),
        in_specs=[
            pl.BlockSpec((N_TOK, D), lambda e: (0, 0)),
            pl.BlockSpec((D, N_EXP), lambda e: (0, 0)),
            pl.BlockSpec((N_TOK, 1), lambda e: (0, 0)),
            pl.BlockSpec((1, D, D), lambda e: (e, 0, 0)),
            pl.BlockSpec((D, D), lambda e: (0, 0)),
        ],
        out_specs=pl.BlockSpec((N_TOK, D), lambda e: (0, 0)),
        out_shape=jax.ShapeDtypeStruct((N_TOK, D), jnp.bfloat16),
        scratch_shapes=[
            pltpu.VMEM((N_TOK, 1), jnp.float32),
            pltpu.VMEM((N_TOK, D), jnp.bfloat16),
            pltpu.VMEM((N_TOK, D), jnp.float32),
        ],
        compiler_params=pltpu.CompilerParams(
            dimension_semantics=("arbitrary",),
        ),
    )(x, router_W, route_idx, expert_W, shared_W)

    return _allreduce(partial)


def _allreduce(partial):
    return pl.pallas_call(
        _allreduce_body,
        out_shape=jax.ShapeDtypeStruct((N_TOK, D), jnp.bfloat16),
        in_specs=[pl.BlockSpec(memory_space=pltpu.VMEM)],
        out_specs=pl.BlockSpec(memory_space=pltpu.VMEM),
        scratch_shapes=[
            pltpu.VMEM((1792, D), jnp.bfloat16),
            pltpu.SemaphoreType.DMA((_N_SEMS,)),
            pltpu.SemaphoreType.DMA((_N_SEMS,)),
        ],
        compiler_params=pltpu.CompilerParams(collective_id=0),
    )(partial)
